# Initial kernel scaffold; baseline (speedup 1.0000x reference)
import jax
import jax.numpy as jnp
from jax.experimental import pallas as pl


def kernel(inputs, pos_table, tok_table):
    raise NotImplementedError("write your pallas kernel here")


# speedup vs baseline: 3.0784x; 3.0784x over previous
import functools

import jax
import jax.numpy as jnp
from jax import lax
from jax.experimental import pallas as pl
from jax.experimental.pallas import tpu as pltpu
from jax.experimental.pallas import tpu_sc as plsc

BATCH = 4096
SEQ_LEN = 200
EMBED_DIM = 64
TOTAL = BATCH * SEQ_LEN

NUM_CORES = 2
NUM_SUBCORES = 16
NUM_WORKERS = NUM_CORES * NUM_SUBCORES
PER_WORKER = TOTAL // NUM_WORKERS

CHUNK = 2 * SEQ_LEN
NUM_CHUNKS = PER_WORKER // CHUNK
IDX_W = 100
IDX_ROWS = CHUNK // IDX_W
LANES = 16
GROUPS = EMBED_DIM // LANES


def kernel(inputs, pos_table, tok_table):
    idx2d = inputs.reshape(TOTAL // IDX_W, IDX_W).astype(jnp.int32)

    mesh = plsc.VectorSubcoreMesh(core_axis_name="c", subcore_axis_name="s")

    @functools.partial(
        pl.kernel,
        out_type=jax.ShapeDtypeStruct((TOTAL, EMBED_DIM), jnp.float32),
        mesh=mesh,
        scratch_types=[
            pltpu.VMEM((IDX_ROWS, IDX_W), jnp.int32),
            pltpu.VMEM((CHUNK, EMBED_DIM), jnp.float32),
            pltpu.VMEM((SEQ_LEN, EMBED_DIM), jnp.float32),
        ],
        compiler_params=pltpu.CompilerParams(use_tc_tiling_on_sc=False),
    )
    def embed(idx_hbm, pos_hbm, tok_hbm, out_hbm, idx_v, rows_v, pos_v):
        wid = lax.axis_index("s") * NUM_CORES + lax.axis_index("c")
        row_base = wid * PER_WORKER
        idx_base = wid * (PER_WORKER // IDX_W)

        pltpu.sync_copy(pos_hbm, pos_v)

        @pl.loop(0, NUM_CHUNKS)
        def _(c):
            start = row_base + c * CHUNK
            pltpu.sync_copy(idx_hbm.at[pl.ds(idx_base + c * IDX_ROWS, IDX_ROWS)],
                            idx_v)
            for j in range(IDX_ROWS):
                pltpu.sync_copy(tok_hbm.at[idx_v.at[j]],
                                rows_v.at[pl.ds(j * IDX_W, IDX_W)])

            @pl.loop(0, SEQ_LEN)
            def _(l):
                for g in range(GROUPS):
                    seg = pl.ds(g * LANES, LANES)
                    pv = pos_v[l, seg]
                    plsc.addupdate(rows_v.at[l, seg], pv)
                    plsc.addupdate(rows_v.at[SEQ_LEN + l, seg], pv)

            pltpu.sync_copy(rows_v, out_hbm.at[pl.ds(start, CHUNK)])

    out = embed(idx2d, pos_table, tok_table)
    return out.reshape(BATCH, SEQ_LEN, EMBED_DIM)
